# Initial kernel scaffold; baseline (speedup 1.0000x reference)
import jax
import jax.numpy as jnp
from jax.experimental import pallas as pl


def kernel(inputs, embeddings):
    raise NotImplementedError("write your pallas kernel here")


# speedup vs baseline: 3.8519x; 3.8519x over previous
import functools

import jax
import jax.numpy as jnp
from jax import lax
from jax.experimental import pallas as pl
from jax.experimental.pallas import tpu as pltpu
from jax.experimental.pallas import tpu_sc as plsc


def _glyph_gather(idx, table, B, L, S):
    info = plsc.get_sparse_core_info()
    NC, NS = info.num_cores, info.num_subcores
    NW = NC * NS
    assert B % NW == 0
    bpw = B // NW
    P = L * S
    NCH = P // 128

    mesh = plsc.VectorSubcoreMesh(core_axis_name="c", subcore_axis_name="s")

    @functools.partial(
        pl.kernel,
        mesh=mesh,
        out_type=jax.ShapeDtypeStruct((B, P, S), jnp.float32),
        compiler_params=pltpu.CompilerParams(use_tc_tiling_on_sc=False),
        scratch_types=[
            pltpu.VMEM((bpw, L), jnp.int32),
            pltpu.VMEM((bpw, NCH, 128), jnp.int32),
            pltpu.VMEM((2, P, S), jnp.float32),
            pltpu.SemaphoreType.DMA,
            pltpu.SemaphoreType.DMA,
            pltpu.SemaphoreType.DMA,
        ],
    )
    def k(idx_hbm, table_hbm, out_hbm, idx_v, ilist_v, t_v, gsem, wsem0, wsem1):
        wid = lax.axis_index("s") * NC + lax.axis_index("c")
        base = wid * bpw
        pltpu.sync_copy(idx_hbm.at[pl.ds(base, bpw)], idx_v)

        def build(i, _):
            for h in range(L // 16):
                a = idx_v[i, pl.ds(h * 16, 16)] * S
                for j in range(NCH):
                    for qq in range(128 // 16 // (L // 16)):
                        q = qq * (L // 16) + h
                        r = (j * 128 + q * 16) // L
                        ilist_v[i, j, pl.ds(q * 16, 16)] = a + r
            return 0
        lax.fori_loop(0, bpw, build, 0)

        def gather_into(i, buf):
            cps = [
                pltpu.async_copy(
                    table_hbm.at[ilist_v.at[i, j]],
                    t_v.at[buf, pl.ds(j * 128, 128)],
                    gsem,
                )
                for j in range(NCH)
            ]
            return cps

        def drain(cps):
            for cp in cps:
                cp.wait()

        def loop(ii, carry):
            i0 = ii * 2
            c0 = gather_into(i0, 0)
            drain(c0)
            w0 = pltpu.async_copy(t_v.at[0], out_hbm.at[base + i0], wsem0)
            c1 = gather_into(i0 + 1, 1)
            drain(c1)
            w0.wait()
            w1 = pltpu.async_copy(t_v.at[1], out_hbm.at[base + i0 + 1], wsem1)
            w1.wait()
            return carry
        lax.fori_loop(0, bpw // 2, loop, 0)

    return k(idx, table)


def kernel(inputs, embeddings):
    B, L = inputs.shape
    V, S, S2, C = embeddings.shape
    idx = inputs.astype(jnp.int32)
    table = embeddings.reshape(V * S, S2)
    out = _glyph_gather(idx, table, B, L, S)
    return out.reshape(B, S, L * S2, 1)


# speedup vs baseline: 6.5165x; 1.6918x over previous
import functools

import jax
import jax.numpy as jnp
from jax import lax
from jax.experimental import pallas as pl
from jax.experimental.pallas import tpu as pltpu
from jax.experimental.pallas import tpu_sc as plsc


def _glyph_gather(ids, table, B, L, S):
    info = plsc.get_sparse_core_info()
    NC, NS = info.num_cores, info.num_subcores
    NW = NC * NS
    assert B % NW == 0 and (B // NW) % 2 == 0
    bpw = B // NW
    D = S * S
    H = S // 2

    mesh = plsc.VectorSubcoreMesh(core_axis_name="c", subcore_axis_name="s")

    @functools.partial(
        pl.kernel,
        mesh=mesh,
        out_type=jax.ShapeDtypeStruct((B, S, L * S), jnp.float32),
        compiler_params=pltpu.CompilerParams(use_tc_tiling_on_sc=False),
        scratch_types=[
            pltpu.VMEM((bpw * L // 128, 128), jnp.int32),
            pltpu.VMEM((2, L, D), jnp.float32),
            pltpu.VMEM((H, L * S), jnp.float32),
            pltpu.SemaphoreType.DMA,
            pltpu.SemaphoreType.DMA,
        ],
    )
    def k(ids_hbm, table_hbm, out_hbm, ids_v, t_v, u_v, gsem0, gsem1):
        wid = lax.axis_index("s") * NC + lax.axis_index("c")
        base = wid * bpw
        nrows = bpw * L // 128
        pltpu.sync_copy(ids_hbm.at[pl.ds(wid * nrows, nrows)], ids_v)

        def fire(i, buf, sem):
            p = i * L
            pltpu.async_copy(
                table_hbm.at[ids_v.at[p // 128, pl.ds(p % 128, L)]],
                t_v.at[buf], sem)

        def drained(buf, sem):
            pltpu.make_async_copy(
                table_hbm.at[pl.ds(0, L)], t_v.at[buf], sem).wait()

        def emit(i, buf):
            for h in range(2):
                def tbody(hr, _):
                    for l in range(L):
                        for cc in range(S // 16):
                            u_v[hr, pl.ds(l * S + cc * 16, 16)] = t_v[
                                buf, l,
                                pl.ds((h * H + hr) * S + cc * 16, 16)]
                    return 0
                lax.fori_loop(0, H, tbody, 0)
                pltpu.sync_copy(
                    u_v, out_hbm.at[base + i, pl.ds(h * H, H)])

        def loop(ii, carry):
            i0 = ii * 2
            fire(i0 + 1, 1, gsem1)
            drained(0, gsem0)
            emit(i0, 0)

            @pl.when(ii + 1 < bpw // 2)
            def _():
                fire(i0 + 2, 0, gsem0)
            drained(1, gsem1)
            emit(i0 + 1, 1)
            return carry

        fire(0, 0, gsem0)
        lax.fori_loop(0, bpw // 2, loop, 0)

    return k(ids, table)


def kernel(inputs, embeddings):
    B, L = inputs.shape
    V, S, S2, C = embeddings.shape
    ids = inputs.astype(jnp.int32).reshape(B * L // 128, 128)
    table = embeddings.reshape(V, S * S2)
    out = _glyph_gather(ids, table, B, L, S)
    return out.reshape(B, S, L * S2, 1)
